# Initial kernel scaffold; baseline (speedup 1.0000x reference)
import jax
import jax.numpy as jnp
from jax.experimental import pallas as pl


def kernel(x, onehot0, edge_index, batch_sample_indices, n_sample_nodes, adj0, W_lin, att_l, att_r, bias, conv1_w, conv1_b, conv2_w, conv2_b, lin16_w, lin16_b):
    raise NotImplementedError("write your pallas kernel here")


# speedup vs baseline: 30.1336x; 30.1336x over previous
import functools

import jax
import jax.numpy as jnp
from jax import lax
from jax.experimental import pallas as pl
from jax.experimental.pallas import tpu as pltpu
from jax.experimental.pallas import tpu_sc as plsc

N = 10000
E = 320000
D = 128
L = 32
H = 4
C = 32
OHC = 8
GW = 176
NPAD = 10240


def _sort32_sublanes(v):
    W = v.shape[1]
    i = lax.broadcasted_iota(jnp.int32, (32, W), 0)
    k = 2
    while k <= 32:
        j = k // 2
        while j >= 1:
            parts = []
            for b in range(0, 32, 2 * j):
                parts.append(v[b + j:b + 2 * j])
                parts.append(v[b:b + j])
            p = jnp.concatenate(parts, axis=0)
            take_min = ((i & j) == 0) == ((i & k) == 0)
            v = jnp.where(take_min, jnp.minimum(v, p), jnp.maximum(v, p))
            j //= 2
        k *= 2
    return v


def _shift_down(v):
    return jnp.concatenate([jnp.zeros_like(v[0:1]), v[:-1]], axis=0)


def _shift_up(v):
    return jnp.concatenate([v[1:], jnp.zeros_like(v[0:1])], axis=0)


def _pipe_body(oh_ref, w1, b1, w2, b2, lw, lb, out_ref):
    v = oh_ref[...]
    v = _sort32_sublanes(v)
    vp = _shift_down(v)
    vn = _shift_up(v)
    y1 = []
    for c in range(8):
        y = w1[c, 0, 0] * vp + w1[c, 0, 1] * v + w1[c, 0, 2] * vn + b1[c]
        y1.append(jnp.maximum(y, 0.0))
    y1p = [_shift_down(y) for y in y1]
    y1n = [_shift_up(y) for y in y1]
    feat = []
    for o in range(16):
        acc = b2[o]
        for c in range(8):
            acc = acc + (w2[o, c, 0] * y1p[c] + w2[o, c, 1] * y1[c]
                         + w2[o, c, 2] * y1n[c])
        acc = jnp.maximum(acc, 0.0)
        feat.append(jnp.mean(acc, axis=0, keepdims=True))
    for jj in range(OHC):
        s = None
        for o in range(16):
            t = lw[jj, o] * feat[o]
            s = t if s is None else s + t
        out_ref[jj:jj + 1, :] = s + lb[jj]


def _onehot_pipe(ohT, conv1_w, conv1_b, conv2_w, conv2_b, lin16_w, lin16_b):
    WB = 2048
    grid = NPAD // WB
    smem = pl.BlockSpec(memory_space=pltpu.SMEM)
    return pl.pallas_call(
        _pipe_body,
        grid=(grid,),
        in_specs=[pl.BlockSpec((32, WB), lambda i: (0, i)),
                  smem, smem, smem, smem, smem, smem],
        out_specs=pl.BlockSpec((OHC, WB), lambda i: (0, i)),
        out_shape=jax.ShapeDtypeStruct((OHC, NPAD), jnp.float32),
    )(ohT, conv1_w, conv1_b, conv2_w, conv2_b, lin16_w, lin16_b)


BN = 1000


def _lin_body(x_ref, ohf_ref, oh0_ref, wxT_ref, woT_ref, attl_ref, attr_ref,
              g_ref, ar_ref):
    xh = (jnp.dot(x_ref[...], wxT_ref[...], preferred_element_type=jnp.float32)
          + jnp.dot(ohf_ref[...], woT_ref[...],
                    preferred_element_type=jnp.float32))
    tl = xh * attl_ref[...]
    tr = xh * attr_ref[...]
    al = jnp.concatenate(
        [jnp.sum(tl[:, h * 32:(h + 1) * 32], axis=1, keepdims=True)
         for h in range(4)], axis=1)
    ar = jnp.concatenate(
        [jnp.sum(tr[:, h * 32:(h + 1) * 32], axis=1, keepdims=True)
         for h in range(4)], axis=1)
    z12 = jnp.zeros((xh.shape[0], 12), jnp.float32)
    g_ref[...] = jnp.concatenate([xh, oh0_ref[...], al, z12], axis=1)
    ar_ref[...] = jnp.concatenate([ar, z12], axis=1)


def _build_tables(x, oh_feat, onehot0, wxT, woT, attl, attr):
    grid = N // BN
    return pl.pallas_call(
        _lin_body,
        grid=(grid,),
        in_specs=[pl.BlockSpec((BN, D), lambda i: (i, 0)),
                  pl.BlockSpec((BN, OHC), lambda i: (i, 0)),
                  pl.BlockSpec((BN, L), lambda i: (i, 0)),
                  pl.BlockSpec((D, D), lambda i: (0, 0)),
                  pl.BlockSpec((OHC, D), lambda i: (0, 0)),
                  pl.BlockSpec((1, D), lambda i: (0, 0)),
                  pl.BlockSpec((1, D), lambda i: (0, 0))],
        out_specs=[pl.BlockSpec((BN, GW), lambda i: (i, 0)),
                   pl.BlockSpec((BN, 16), lambda i: (i, 0))],
        out_shape=[jax.ShapeDtypeStruct((N, GW), jnp.float32),
                   jax.ShapeDtypeStruct((N, 16), jnp.float32)],
    )(x, oh_feat, onehot0, wxT, woT, attl, attr)


NSC = 2
NTEC = 16
EPW = E // (NSC * NTEC)
CHW = 40
NCH = EPW // CHW
ACC_ROWS = 10240
ROWS_PT = ACC_ROWS // NTEC


def _bcast16(v, h):
    idx = jnp.full((16, 1), h, dtype=jnp.int32)
    return lax.gather(
        v, idx,
        lax.GatherDimensionNumbers(offset_dims=(), collapsed_slice_dims=(0,),
                                   start_index_map=(0,)),
        (1,), mode=lax.GatherScatterMode.PROMISE_IN_BOUNDS)


def _sc_edge_body(g_hbm, ar_hbm, srcI_hbm, dstI_hbm, p_hbm,
                  acc, idxs, idxd, gbuf, abuf, obuf, semg, sema):
    cid = lax.axis_index("c")
    sid = lax.axis_index("s")
    wid = cid * NTEC + sid

    def _zrow(r, _):
        for col in range(GW // 16):
            obuf[r, pl.ds(col * 16, 16)] = jnp.zeros((16,), jnp.float32)
        return 0
    lax.fori_loop(0, CHW, _zrow, 0)
    base = sid * ROWS_PT
    for t in range(ROWS_PT // CHW):
        pltpu.sync_copy(obuf.at[pl.ds(0, CHW)],
                        acc.at[pl.ds(base + t * CHW, CHW)])
    plsc.subcore_barrier()

    def chunk_body(ci, _):
        pltpu.sync_copy(srcI_hbm.at[wid, ci], idxs)
        pltpu.sync_copy(dstI_hbm.at[wid, ci], idxd)
        cp_g = pltpu.async_copy(g_hbm.at[idxs], gbuf, semg)
        cp_a = pltpu.async_copy(ar_hbm.at[idxd], abuf, sema)
        cp_g.wait()
        cp_a.wait()

        def edge_body(e, _):
            t = gbuf[e, pl.ds(160, 16)] + abuf[e, pl.ds(0, 16)]
            t = jnp.where(t >= 0, t, 0.2 * t)
            w = jnp.exp(t)
            for h in range(H):
                wh = _bcast16(w, h)
                for v2 in range(2):
                    off = h * 32 + v2 * 16
                    obuf[e, pl.ds(off, 16)] = gbuf[e, pl.ds(off, 16)] * wh
            obuf[e, pl.ds(128, 16)] = gbuf[e, pl.ds(128, 16)]
            obuf[e, pl.ds(144, 16)] = gbuf[e, pl.ds(144, 16)]
            obuf[e, pl.ds(160, 16)] = w
            return 0
        lax.fori_loop(0, CHW, edge_body, 0)
        pltpu.sync_copy(obuf, acc.at[idxd], add=True)
        return 0
    lax.fori_loop(0, NCH, chunk_body, 0)
    plsc.subcore_barrier()

    pltpu.sync_copy(acc.at[pl.ds(base, ROWS_PT)],
                    p_hbm.at[cid, pl.ds(base, ROWS_PT)])


def _edge_phase_sc(G, AR, edge_index, interpret=False):
    srcI = edge_index[0].reshape(NSC * NTEC, NCH, CHW)
    dstI = edge_index[1].reshape(NSC * NTEC, NCH, CHW)
    mesh = plsc.VectorSubcoreMesh(core_axis_name="c", subcore_axis_name="s",
                                  num_cores=NSC, num_subcores=NTEC)
    f = pl.kernel(
        _sc_edge_body,
        out_type=jax.ShapeDtypeStruct((NSC, ACC_ROWS, GW), jnp.float32),
        mesh=mesh,
        scratch_types=[
            pltpu.VMEM_SHARED((ACC_ROWS, GW), jnp.float32),
            pltpu.VMEM((CHW,), jnp.int32),
            pltpu.VMEM((CHW,), jnp.int32),
            pltpu.VMEM((CHW, GW), jnp.float32),
            pltpu.VMEM((CHW, 16), jnp.float32),
            pltpu.VMEM((CHW, GW), jnp.float32),
            pltpu.SemaphoreType.DMA,
            pltpu.SemaphoreType.DMA,
        ],
        compiler_params=pltpu.CompilerParams(use_tc_tiling_on_sc=False),
        interpret=interpret,
    )
    return f(G, AR, srcI, dstI)


def _edge_phase_xla(G, AR, edge_index):
    src = edge_index[0]
    dst = edge_index[1]
    al = G[:, 160:164]
    ar4 = AR[:, :4]
    t = jnp.take(al, src, axis=0) + jnp.take(ar4, dst, axis=0)
    t = jnp.where(t >= 0, t, 0.2 * t)
    w = jnp.exp(t)
    wrep = jnp.repeat(w, 32, axis=1)
    gs = jnp.take(G, src, axis=0)
    S1 = jax.ops.segment_sum(gs[:, :128] * wrep, dst, num_segments=N)
    OH = jax.ops.segment_sum(gs[:, 128:160], dst, num_segments=N)
    S0 = jax.ops.segment_sum(w, dst, num_segments=N)
    P0 = jnp.concatenate([S1, OH, S0, jnp.zeros((N, 12))], axis=1)
    return jnp.stack([P0, jnp.zeros_like(P0)])


def _comb_body(p0_ref, p1_ref, oh0_ref, bias_ref, xout_ref, ohout_ref):
    S = p0_ref[0] + p1_ref[0]
    chunks = []
    for h in range(4):
        s0 = S[:, 160 + h:161 + h]
        chunks.append(S[:, h * 32:(h + 1) * 32] / (s0 + 1e-16))
    xout_ref[...] = jnp.concatenate(chunks, axis=1) + bias_ref[...]
    ohout_ref[...] = S[:, 128:160] + oh0_ref[...]


def _combine(P, onehot0, bias):
    grid = N // BN
    return pl.pallas_call(
        _comb_body,
        grid=(grid,),
        in_specs=[pl.BlockSpec((1, BN, GW), lambda i: (0, i, 0)),
                  pl.BlockSpec((1, BN, GW), lambda i: (0, i, 0)),
                  pl.BlockSpec((BN, L), lambda i: (i, 0)),
                  pl.BlockSpec((1, D), lambda i: (0, 0))],
        out_specs=[pl.BlockSpec((BN, D), lambda i: (i, 0)),
                   pl.BlockSpec((BN, L), lambda i: (i, 0))],
        out_shape=[jax.ShapeDtypeStruct((N, D), jnp.float32),
                   jax.ShapeDtypeStruct((N, L), jnp.float32)],
    )(P[0:1], P[1:2], onehot0, bias)


def kernel(x, onehot0, edge_index, batch_sample_indices, n_sample_nodes, adj0,
           W_lin, att_l, att_r, bias, conv1_w, conv1_b, conv2_w, conv2_b,
           lin16_w, lin16_b):
    f32 = jnp.float32
    ohT = jnp.zeros((L, NPAD), f32).at[:, :N].set(onehot0.T)
    wxT = W_lin[:, :D].T
    woT = W_lin[:, D:].T
    attl = att_l.reshape(1, H * C)
    attr = att_r.reshape(1, H * C)

    ohfT = _onehot_pipe(ohT, conv1_w, conv1_b, conv2_w, conv2_b,
                        lin16_w, lin16_b)
    oh_feat = ohfT.T[:N]

    G, AR = _build_tables(x, oh_feat, onehot0, wxT, woT, attl, attr)

    P = _edge_phase_sc(G, AR, edge_index)

    x_out, new_oh = _combine(P, onehot0, bias.reshape(1, D))
    return (x_out, new_oh)


# speedup vs baseline: 37.8705x; 1.2568x over previous
import functools

import jax
import jax.numpy as jnp
from jax import lax
from jax.experimental import pallas as pl
from jax.experimental.pallas import tpu as pltpu
from jax.experimental.pallas import tpu_sc as plsc

N = 10000
E = 320000
D = 128
L = 32
H = 4
C = 32
OHC = 8
GW = 176
NPAD = 10240


def _sort32_sublanes(v):
    W = v.shape[1]
    i = lax.broadcasted_iota(jnp.int32, (32, W), 0)
    k = 2
    while k <= 32:
        j = k // 2
        while j >= 1:
            parts = []
            for b in range(0, 32, 2 * j):
                parts.append(v[b + j:b + 2 * j])
                parts.append(v[b:b + j])
            p = jnp.concatenate(parts, axis=0)
            take_min = ((i & j) == 0) == ((i & k) == 0)
            v = jnp.where(take_min, jnp.minimum(v, p), jnp.maximum(v, p))
            j //= 2
        k *= 2
    return v


def _shift_down(v):
    return jnp.concatenate([jnp.zeros_like(v[0:1]), v[:-1]], axis=0)


def _shift_up(v):
    return jnp.concatenate([v[1:], jnp.zeros_like(v[0:1])], axis=0)


def _pipe_body(oh_ref, w1, b1, w2, b2, lw, lb, out_ref):
    v = oh_ref[...]
    v = _sort32_sublanes(v)
    vp = _shift_down(v)
    vn = _shift_up(v)
    y1 = []
    for c in range(8):
        y = w1[c, 0, 0] * vp + w1[c, 0, 1] * v + w1[c, 0, 2] * vn + b1[c]
        y1.append(jnp.maximum(y, 0.0))
    y1p = [_shift_down(y) for y in y1]
    y1n = [_shift_up(y) for y in y1]
    feat = []
    for o in range(16):
        acc = b2[o]
        for c in range(8):
            acc = acc + (w2[o, c, 0] * y1p[c] + w2[o, c, 1] * y1[c]
                         + w2[o, c, 2] * y1n[c])
        acc = jnp.maximum(acc, 0.0)
        feat.append(jnp.mean(acc, axis=0, keepdims=True))
    for jj in range(OHC):
        s = None
        for o in range(16):
            t = lw[jj, o] * feat[o]
            s = t if s is None else s + t
        out_ref[jj:jj + 1, :] = s + lb[jj]


def _onehot_pipe(ohT, conv1_w, conv1_b, conv2_w, conv2_b, lin16_w, lin16_b):
    WB = 2048
    grid = NPAD // WB
    smem = pl.BlockSpec(memory_space=pltpu.SMEM)
    return pl.pallas_call(
        _pipe_body,
        grid=(grid,),
        in_specs=[pl.BlockSpec((32, WB), lambda i: (0, i)),
                  smem, smem, smem, smem, smem, smem],
        out_specs=pl.BlockSpec((OHC, WB), lambda i: (0, i)),
        out_shape=jax.ShapeDtypeStruct((OHC, NPAD), jnp.float32),
    )(ohT, conv1_w, conv1_b, conv2_w, conv2_b, lin16_w, lin16_b)


BN = 1000


def _lin_body(x_ref, ohf_ref, oh0_ref, wxT_ref, woT_ref, attl_ref, attr_ref,
              g_ref, ar_ref):
    xh = (jnp.dot(x_ref[...], wxT_ref[...], preferred_element_type=jnp.float32)
          + jnp.dot(ohf_ref[...], woT_ref[...],
                    preferred_element_type=jnp.float32))
    tl = xh * attl_ref[...]
    tr = xh * attr_ref[...]
    al = jnp.concatenate(
        [jnp.sum(tl[:, h * 32:(h + 1) * 32], axis=1, keepdims=True)
         for h in range(4)], axis=1)
    ar = jnp.concatenate(
        [jnp.sum(tr[:, h * 32:(h + 1) * 32], axis=1, keepdims=True)
         for h in range(4)], axis=1)
    z12 = jnp.zeros((xh.shape[0], 12), jnp.float32)
    g_ref[...] = jnp.concatenate([xh, oh0_ref[...], al, z12], axis=1)
    ar_ref[...] = jnp.concatenate([ar, z12], axis=1)


def _build_tables(x, oh_feat, onehot0, wxT, woT, attl, attr):
    grid = N // BN
    return pl.pallas_call(
        _lin_body,
        grid=(grid,),
        in_specs=[pl.BlockSpec((BN, D), lambda i: (i, 0)),
                  pl.BlockSpec((BN, OHC), lambda i: (i, 0)),
                  pl.BlockSpec((BN, L), lambda i: (i, 0)),
                  pl.BlockSpec((D, D), lambda i: (0, 0)),
                  pl.BlockSpec((OHC, D), lambda i: (0, 0)),
                  pl.BlockSpec((1, D), lambda i: (0, 0)),
                  pl.BlockSpec((1, D), lambda i: (0, 0))],
        out_specs=[pl.BlockSpec((BN, GW), lambda i: (i, 0)),
                   pl.BlockSpec((BN, 16), lambda i: (i, 0))],
        out_shape=[jax.ShapeDtypeStruct((N, GW), jnp.float32),
                   jax.ShapeDtypeStruct((N, 16), jnp.float32)],
    )(x, oh_feat, onehot0, wxT, woT, attl, attr)


NSC = 2
NTEC = 16
EPW = E // (NSC * NTEC)
CHW = 40
NCH = EPW // CHW
KCH = 25
ACC_ROWS = 10240
ROWS_PT = ACC_ROWS // NTEC


def _bcast16(v, h):
    idx = jnp.full((16, 1), h, dtype=jnp.int32)
    return lax.gather(
        v, idx,
        lax.GatherDimensionNumbers(offset_dims=(), collapsed_slice_dims=(0,),
                                   start_index_map=(0,)),
        (1,), mode=lax.GatherScatterMode.PROMISE_IN_BOUNDS)


def _sc_edge_body(g_hbm, ar_hbm, srcI_hbm, dstI_hbm, p_hbm,
                  acc, idxs, idxd, gbuf, abuf, obuf, semg, sema):
    cid = lax.axis_index("c")
    sid = lax.axis_index("s")
    wid = cid * NTEC + sid

    def _zrow(r, _):
        for col in range(GW // 16):
            obuf[r, pl.ds(col * 16, 16)] = jnp.zeros((16,), jnp.float32)
        return 0
    lax.fori_loop(0, CHW, _zrow, 0)
    base = sid * ROWS_PT
    for t in range(ROWS_PT // CHW):
        pltpu.sync_copy(obuf.at[pl.ds(0, CHW)],
                        acc.at[pl.ds(base + t * CHW, CHW)])
    plsc.subcore_barrier()

    def chunk_body(ci, _):
        kb = ci // KCH
        kj = ci % KCH

        @pl.when(kj == 0)
        def _stage_indices():
            pltpu.sync_copy(srcI_hbm.at[wid, pl.ds(kb * KCH, KCH)], idxs)
            pltpu.sync_copy(dstI_hbm.at[wid, pl.ds(kb * KCH, KCH)], idxd)

        cp_g = pltpu.async_copy(g_hbm.at[idxs.at[kj]], gbuf, semg)
        cp_a = pltpu.async_copy(ar_hbm.at[idxd.at[kj]], abuf, sema)
        cp_g.wait()
        cp_a.wait()

        def edge_body(e, _):
            t = gbuf[e, pl.ds(160, 16)] + abuf[e, pl.ds(0, 16)]
            t = jnp.where(t >= 0, t, 0.2 * t)
            w = jnp.exp(t)
            for h in range(H):
                wh = _bcast16(w, h)
                for v2 in range(2):
                    off = h * 32 + v2 * 16
                    obuf[e, pl.ds(off, 16)] = gbuf[e, pl.ds(off, 16)] * wh
            obuf[e, pl.ds(128, 16)] = gbuf[e, pl.ds(128, 16)]
            obuf[e, pl.ds(144, 16)] = gbuf[e, pl.ds(144, 16)]
            obuf[e, pl.ds(160, 16)] = w
            return 0
        lax.fori_loop(0, CHW, edge_body, 0)
        pltpu.sync_copy(obuf, acc.at[idxd.at[kj]], add=True)
        return 0
    lax.fori_loop(0, NCH, chunk_body, 0)
    plsc.subcore_barrier()

    pltpu.sync_copy(acc.at[pl.ds(base, ROWS_PT)],
                    p_hbm.at[cid, pl.ds(base, ROWS_PT)])


def _edge_phase_sc(G, AR, edge_index, interpret=False):
    srcI = edge_index[0].reshape(NSC * NTEC, NCH, CHW)
    dstI = edge_index[1].reshape(NSC * NTEC, NCH, CHW)
    mesh = plsc.VectorSubcoreMesh(core_axis_name="c", subcore_axis_name="s",
                                  num_cores=NSC, num_subcores=NTEC)
    f = pl.kernel(
        _sc_edge_body,
        out_type=jax.ShapeDtypeStruct((NSC, ACC_ROWS, GW), jnp.float32),
        mesh=mesh,
        scratch_types=[
            pltpu.VMEM_SHARED((ACC_ROWS, GW), jnp.float32),
            pltpu.VMEM((KCH, CHW), jnp.int32),
            pltpu.VMEM((KCH, CHW), jnp.int32),
            pltpu.VMEM((CHW, GW), jnp.float32),
            pltpu.VMEM((CHW, 16), jnp.float32),
            pltpu.VMEM((CHW, GW), jnp.float32),
            pltpu.SemaphoreType.DMA,
            pltpu.SemaphoreType.DMA,
        ],
        compiler_params=pltpu.CompilerParams(use_tc_tiling_on_sc=False),
        interpret=interpret,
    )
    return f(G, AR, srcI, dstI)


def _edge_phase_xla(G, AR, edge_index):
    src = edge_index[0]
    dst = edge_index[1]
    al = G[:, 160:164]
    ar4 = AR[:, :4]
    t = jnp.take(al, src, axis=0) + jnp.take(ar4, dst, axis=0)
    t = jnp.where(t >= 0, t, 0.2 * t)
    w = jnp.exp(t)
    wrep = jnp.repeat(w, 32, axis=1)
    gs = jnp.take(G, src, axis=0)
    S1 = jax.ops.segment_sum(gs[:, :128] * wrep, dst, num_segments=N)
    OH = jax.ops.segment_sum(gs[:, 128:160], dst, num_segments=N)
    S0 = jax.ops.segment_sum(w, dst, num_segments=N)
    P0 = jnp.concatenate([S1, OH, S0, jnp.zeros((N, 12))], axis=1)
    return jnp.stack([P0, jnp.zeros_like(P0)])


def _comb_body(p0_ref, p1_ref, oh0_ref, bias_ref, xout_ref, ohout_ref):
    S = p0_ref[0] + p1_ref[0]
    chunks = []
    for h in range(4):
        s0 = S[:, 160 + h:161 + h]
        chunks.append(S[:, h * 32:(h + 1) * 32] / (s0 + 1e-16))
    xout_ref[...] = jnp.concatenate(chunks, axis=1) + bias_ref[...]
    ohout_ref[...] = S[:, 128:160] + oh0_ref[...]


def _combine(P, onehot0, bias):
    grid = N // BN
    return pl.pallas_call(
        _comb_body,
        grid=(grid,),
        in_specs=[pl.BlockSpec((1, BN, GW), lambda i: (0, i, 0)),
                  pl.BlockSpec((1, BN, GW), lambda i: (0, i, 0)),
                  pl.BlockSpec((BN, L), lambda i: (i, 0)),
                  pl.BlockSpec((1, D), lambda i: (0, 0))],
        out_specs=[pl.BlockSpec((BN, D), lambda i: (i, 0)),
                   pl.BlockSpec((BN, L), lambda i: (i, 0))],
        out_shape=[jax.ShapeDtypeStruct((N, D), jnp.float32),
                   jax.ShapeDtypeStruct((N, L), jnp.float32)],
    )(P[0:1], P[1:2], onehot0, bias)


def kernel(x, onehot0, edge_index, batch_sample_indices, n_sample_nodes, adj0,
           W_lin, att_l, att_r, bias, conv1_w, conv1_b, conv2_w, conv2_b,
           lin16_w, lin16_b):
    f32 = jnp.float32
    ohT = jnp.zeros((L, NPAD), f32).at[:, :N].set(onehot0.T)
    wxT = W_lin[:, :D].T
    woT = W_lin[:, D:].T
    attl = att_l.reshape(1, H * C)
    attr = att_r.reshape(1, H * C)

    ohfT = _onehot_pipe(ohT, conv1_w, conv1_b, conv2_w, conv2_b,
                        lin16_w, lin16_b)
    oh_feat = ohfT.T[:N]

    G, AR = _build_tables(x, oh_feat, onehot0, wxT, woT, attl, attr)

    P = _edge_phase_sc(G, AR, edge_index)

    x_out, new_oh = _combine(P, onehot0, bias.reshape(1, D))
    return (x_out, new_oh)


# speedup vs baseline: 37.8888x; 1.0005x over previous
import jax
import jax.numpy as jnp
from jax import lax
from jax.experimental import pallas as pl
from jax.experimental.pallas import tpu as pltpu
from jax.experimental.pallas import tpu_sc as plsc

N = 10000
E = 320000
D = 128
L = 32
H = 4
C = 32
OHC = 8
GW = 176
NPAD = 10240


def _sort32_sublanes(v):
    W = v.shape[1]
    i = lax.broadcasted_iota(jnp.int32, (32, W), 0)
    k = 2
    while k <= 32:
        j = k // 2
        while j >= 1:
            parts = []
            for b in range(0, 32, 2 * j):
                parts.append(v[b + j:b + 2 * j])
                parts.append(v[b:b + j])
            p = jnp.concatenate(parts, axis=0)
            take_min = ((i & j) == 0) == ((i & k) == 0)
            v = jnp.where(take_min, jnp.minimum(v, p), jnp.maximum(v, p))
            j //= 2
        k *= 2
    return v


def _shift_down(v):
    return jnp.concatenate([jnp.zeros_like(v[0:1]), v[:-1]], axis=0)


def _shift_up(v):
    return jnp.concatenate([v[1:], jnp.zeros_like(v[0:1])], axis=0)


def _pipe_body(oh_ref, w1, b1, w2, b2, lw, lb, out_ref):
    v = oh_ref[...]
    v = _sort32_sublanes(v)
    vp = _shift_down(v)
    vn = _shift_up(v)
    y1 = []
    for c in range(8):
        y = w1[c, 0, 0] * vp + w1[c, 0, 1] * v + w1[c, 0, 2] * vn + b1[c]
        y1.append(jnp.maximum(y, 0.0))
    y1p = [_shift_down(y) for y in y1]
    y1n = [_shift_up(y) for y in y1]
    feat = []
    for o in range(16):
        acc = b2[o]
        for c in range(8):
            acc = acc + (w2[o, c, 0] * y1p[c] + w2[o, c, 1] * y1[c]
                         + w2[o, c, 2] * y1n[c])
        acc = jnp.maximum(acc, 0.0)
        feat.append(jnp.mean(acc, axis=0, keepdims=True))
    for jj in range(OHC):
        s = None
        for o in range(16):
            t = lw[jj, o] * feat[o]
            s = t if s is None else s + t
        out_ref[jj:jj + 1, :] = s + lb[jj]


def _onehot_pipe(ohT, conv1_w, conv1_b, conv2_w, conv2_b, lin16_w, lin16_b):
    WB = 2048
    grid = NPAD // WB
    smem = pl.BlockSpec(memory_space=pltpu.SMEM)
    return pl.pallas_call(
        _pipe_body,
        grid=(grid,),
        in_specs=[pl.BlockSpec((32, WB), lambda i: (0, i)),
                  smem, smem, smem, smem, smem, smem],
        out_specs=pl.BlockSpec((OHC, WB), lambda i: (0, i)),
        out_shape=jax.ShapeDtypeStruct((OHC, NPAD), jnp.float32),
    )(ohT, conv1_w, conv1_b, conv2_w, conv2_b, lin16_w, lin16_b)


BN = 1000


def _lin_body(x_ref, ohf_ref, oh0_ref, wxT_ref, woT_ref, attl_ref, attr_ref,
              g_ref, ar_ref):
    xh = (jnp.dot(x_ref[...], wxT_ref[...], preferred_element_type=jnp.float32)
          + jnp.dot(ohf_ref[...], woT_ref[...],
                    preferred_element_type=jnp.float32))
    tl = xh * attl_ref[...]
    tr = xh * attr_ref[...]
    al = jnp.concatenate(
        [jnp.sum(tl[:, h * 32:(h + 1) * 32], axis=1, keepdims=True)
         for h in range(4)], axis=1)
    ar = jnp.concatenate(
        [jnp.sum(tr[:, h * 32:(h + 1) * 32], axis=1, keepdims=True)
         for h in range(4)], axis=1)
    z12 = jnp.zeros((xh.shape[0], 12), jnp.float32)
    g_ref[...] = jnp.concatenate([xh, oh0_ref[...], al, z12], axis=1)
    ar_ref[...] = jnp.concatenate([ar, z12], axis=1)


def _build_tables(x, oh_feat, onehot0, wxT, woT, attl, attr):
    grid = N // BN
    return pl.pallas_call(
        _lin_body,
        grid=(grid,),
        in_specs=[pl.BlockSpec((BN, D), lambda i: (i, 0)),
                  pl.BlockSpec((BN, OHC), lambda i: (i, 0)),
                  pl.BlockSpec((BN, L), lambda i: (i, 0)),
                  pl.BlockSpec((D, D), lambda i: (0, 0)),
                  pl.BlockSpec((OHC, D), lambda i: (0, 0)),
                  pl.BlockSpec((1, D), lambda i: (0, 0)),
                  pl.BlockSpec((1, D), lambda i: (0, 0))],
        out_specs=[pl.BlockSpec((BN, GW), lambda i: (i, 0)),
                   pl.BlockSpec((BN, 16), lambda i: (i, 0))],
        out_shape=[jax.ShapeDtypeStruct((N, GW), jnp.float32),
                   jax.ShapeDtypeStruct((N, 16), jnp.float32)],
    )(x, oh_feat, onehot0, wxT, woT, attl, attr)


NSC = 2
NTEC = 16
EPW = E // (NSC * NTEC)
CHW = 40
NCH = EPW // CHW
KCH = 25
ACC_ROWS = 10240
ROWS_PT = ACC_ROWS // NTEC


def _bcast16(v, h):
    idx = jnp.full((16, 1), h, dtype=jnp.int32)
    return lax.gather(
        v, idx,
        lax.GatherDimensionNumbers(offset_dims=(), collapsed_slice_dims=(0,),
                                   start_index_map=(0,)),
        (1,), mode=lax.GatherScatterMode.PROMISE_IN_BOUNDS)


def _sc_edge_body(g_hbm, ar_hbm, srcI_hbm, dstI_hbm, p_hbm,
                  acc, idxs, idxd, gbuf, abuf, obuf, semg, sema):
    cid = lax.axis_index("c")
    sid = lax.axis_index("s")
    wid = cid * NTEC + sid

    def _zrow(r, _):
        for col in range(GW // 16):
            obuf[r, pl.ds(col * 16, 16)] = jnp.zeros((16,), jnp.float32)
        return 0
    lax.fori_loop(0, CHW, _zrow, 0)
    base = sid * ROWS_PT
    for t in range(ROWS_PT // CHW):
        pltpu.sync_copy(obuf.at[pl.ds(0, CHW)],
                        acc.at[pl.ds(base + t * CHW, CHW)])
    plsc.subcore_barrier()

    def chunk_body(ci, _):
        kb = ci // KCH
        kj = ci % KCH

        @pl.when(kj == 0)
        def _stage_indices():
            pltpu.sync_copy(srcI_hbm.at[wid, pl.ds(kb * KCH, KCH)], idxs)
            pltpu.sync_copy(dstI_hbm.at[wid, pl.ds(kb * KCH, KCH)], idxd)

        cp_g = pltpu.async_copy(g_hbm.at[idxs.at[kj]], gbuf, semg)
        cp_a = pltpu.async_copy(ar_hbm.at[idxd.at[kj]], abuf, sema)
        cp_g.wait()
        cp_a.wait()

        def edge_body(e, _):
            t = gbuf[e, pl.ds(160, 16)] + abuf[e, pl.ds(0, 16)]
            t = jnp.where(t >= 0, t, 0.2 * t)
            w = jnp.exp(t)
            for h in range(H):
                wh = _bcast16(w, h)
                for v2 in range(2):
                    off = h * 32 + v2 * 16
                    obuf[e, pl.ds(off, 16)] = gbuf[e, pl.ds(off, 16)] * wh
            obuf[e, pl.ds(128, 16)] = gbuf[e, pl.ds(128, 16)]
            obuf[e, pl.ds(144, 16)] = gbuf[e, pl.ds(144, 16)]
            obuf[e, pl.ds(160, 16)] = w
            return 0
        lax.fori_loop(0, CHW, edge_body, 0)
        pltpu.sync_copy(obuf, acc.at[idxd.at[kj]], add=True)
        return 0
    lax.fori_loop(0, NCH, chunk_body, 0)
    plsc.subcore_barrier()

    pltpu.sync_copy(acc.at[pl.ds(base, ROWS_PT)],
                    p_hbm.at[cid, pl.ds(base, ROWS_PT)])


def _edge_phase_sc(G, AR, edge_index):
    srcI = edge_index[0].reshape(NSC * NTEC, NCH, CHW)
    dstI = edge_index[1].reshape(NSC * NTEC, NCH, CHW)
    mesh = plsc.VectorSubcoreMesh(core_axis_name="c", subcore_axis_name="s",
                                  num_cores=NSC, num_subcores=NTEC)
    f = pl.kernel(
        _sc_edge_body,
        out_type=jax.ShapeDtypeStruct((NSC, ACC_ROWS, GW), jnp.float32),
        mesh=mesh,
        scratch_types=[
            pltpu.VMEM_SHARED((ACC_ROWS, GW), jnp.float32),
            pltpu.VMEM((KCH, CHW), jnp.int32),
            pltpu.VMEM((KCH, CHW), jnp.int32),
            pltpu.VMEM((CHW, GW), jnp.float32),
            pltpu.VMEM((CHW, 16), jnp.float32),
            pltpu.VMEM((CHW, GW), jnp.float32),
            pltpu.SemaphoreType.DMA,
            pltpu.SemaphoreType.DMA,
        ],
        compiler_params=pltpu.CompilerParams(use_tc_tiling_on_sc=False),
    )
    return f(G, AR, srcI, dstI)


def _comb_body(p0_ref, p1_ref, oh0_ref, bias_ref, xout_ref, ohout_ref):
    S = p0_ref[0] + p1_ref[0]
    chunks = []
    for h in range(4):
        s0 = S[:, 160 + h:161 + h]
        chunks.append(S[:, h * 32:(h + 1) * 32] / (s0 + 1e-16))
    xout_ref[...] = jnp.concatenate(chunks, axis=1) + bias_ref[...]
    ohout_ref[...] = S[:, 128:160] + oh0_ref[...]


def _combine(P, onehot0, bias):
    grid = N // BN
    return pl.pallas_call(
        _comb_body,
        grid=(grid,),
        in_specs=[pl.BlockSpec((1, BN, GW), lambda i: (0, i, 0)),
                  pl.BlockSpec((1, BN, GW), lambda i: (0, i, 0)),
                  pl.BlockSpec((BN, L), lambda i: (i, 0)),
                  pl.BlockSpec((1, D), lambda i: (0, 0))],
        out_specs=[pl.BlockSpec((BN, D), lambda i: (i, 0)),
                   pl.BlockSpec((BN, L), lambda i: (i, 0))],
        out_shape=[jax.ShapeDtypeStruct((N, D), jnp.float32),
                   jax.ShapeDtypeStruct((N, L), jnp.float32)],
    )(P[0:1], P[1:2], onehot0, bias)


def kernel(x, onehot0, edge_index, batch_sample_indices, n_sample_nodes, adj0,
           W_lin, att_l, att_r, bias, conv1_w, conv1_b, conv2_w, conv2_b,
           lin16_w, lin16_b):
    f32 = jnp.float32
    ohT = jnp.zeros((L, NPAD), f32).at[:, :N].set(onehot0.T)
    wxT = W_lin[:, :D].T
    woT = W_lin[:, D:].T
    attl = att_l.reshape(1, H * C)
    attr = att_r.reshape(1, H * C)

    ohfT = _onehot_pipe(ohT, conv1_w, conv1_b, conv2_w, conv2_b,
                        lin16_w, lin16_b)
    oh_feat = ohfT.T[:N]

    G, AR = _build_tables(x, oh_feat, onehot0, wxT, woT, attl, attr)

    P = _edge_phase_sc(G, AR, edge_index)

    x_out, new_oh = _combine(P, onehot0, bias.reshape(1, D))
    return (x_out, new_oh)
